# Initial kernel scaffold; baseline (speedup 1.0000x reference)
import jax
import jax.numpy as jnp
from jax.experimental import pallas as pl


def kernel(x, edge_index, edge_attr, W0, b0, Ws, Wout, bout):
    raise NotImplementedError("write your pallas kernel here")


# speedup vs baseline: 14.8173x; 14.8173x over previous
import functools
import math

import jax
import jax.numpy as jnp
from jax import lax
from jax.experimental import pallas as pl
from jax.experimental.pallas import tpu as pltpu
from jax.experimental.pallas import tpu_sc as plsc

N_NODES = 10000
N_EDGES = 320000
FEAT = 128
N_CLASSES = 64
N_LAYERS = 8
ALPHA_C = 0.1
LAMBDA_C = 0.5

NC = 2
NS = 16
NW = NC * NS
P = 10112
RPS = P // NS
EPW = N_EDGES // NW
CK = 80
NCH = EPW // CK
RS = 4
ES = 6

BN = 512

_SC_PARAMS = pltpu.CompilerParams(needs_layout_passes=False)


def _zero_rows(buf, nrows):
    z = jnp.zeros((16,), jnp.float32)
    ncol = buf.shape[1]

    def body(i, carry):
        for j in range(ncol // 16):
            buf[i, pl.ds(j * 16, 16)] = z
        return carry

    lax.fori_loop(0, nrows, body, None)


def _spmm_body(hh_hbm, row_hbm, col_hbm, ew_hbm, out_hbm,
               rows, ri, ci, ewc, acc, sem_g, sem_s, sem_e):
    c = lax.axis_index("c")
    s = lax.axis_index("s")
    w = c * NS + s
    base = s * RPS
    ebase = w * EPW

    def issue_edge(k, e):
        cb = ebase + k * CK
        pltpu.async_copy(row_hbm.at[pl.ds(cb, CK)], ri[e], sem_e[e])
        pltpu.async_copy(col_hbm.at[pl.ds(cb, CK)], ci[e], sem_e[e])
        pltpu.async_copy(ew_hbm.at[pl.ds(cb, CK)], ewc[e], sem_e[e])

    def wait_edge(e):
        pltpu.make_async_copy(row_hbm.at[pl.ds(ebase, CK)], ri[e],
                              sem_e[e]).wait()
        pltpu.make_async_copy(col_hbm.at[pl.ds(ebase, CK)], ci[e],
                              sem_e[e]).wait()
        pltpu.make_async_copy(ew_hbm.at[pl.ds(ebase, CK)], ewc[e],
                              sem_e[e]).wait()

    def issue_g(r, e):
        pltpu.async_copy(hh_hbm.at[ri[e]], rows[r], sem_g[r])

    def wait_g(r):
        pltpu.make_async_copy(hh_hbm.at[ri[0]], rows[r], sem_g[r]).wait()

    def issue_s(r, e):
        pltpu.async_copy(rows[r], acc.at[ci[e]], sem_s[r], add=True)

    def wait_s(r):
        pltpu.make_async_copy(rows[r], acc.at[ci[0]], sem_s[r]).wait()

    def scale(r, e):
        def group(g, cc):
            for u in range(8):
                i = g * 8 + u
                nrm = plsc.load_gather(ewc[e], [jnp.full((16,), i, jnp.int32)])
                for j in range(FEAT // 16):
                    rows[r][i, pl.ds(j * 16, 16)] = \
                        rows[r][i, pl.ds(j * 16, 16)] * nrm
            return cc

        lax.fori_loop(0, CK // 8, group, None)

    def step(k, u):
        m4, m6 = u % RS, u % ES
        wait_g(m4)
        scale(m4, m6)
        issue_s(m4, m6)
        static = isinstance(k, int)
        if not static or k >= 2:
            wait_s((u - 2) % RS)
        if not static or k + 4 <= NCH - 1:
            issue_edge(k + 4, (u + 4) % ES)
        if not static or k + 2 <= NCH - 1:
            wait_edge((u + 2) % ES)
            issue_g((u + 2) % RS, (u + 2) % ES)

    for k in range(4):
        issue_edge(k, k)

    _zero_rows(rows[0], CK)
    for t in range(7):
        pltpu.sync_copy(rows[0], acc.at[pl.ds(base + t * CK, CK)])
    pltpu.sync_copy(rows[0].at[pl.ds(0, RPS - 7 * CK)],
                    acc.at[pl.ds(base + 7 * CK, RPS - 7 * CK)])
    plsc.subcore_barrier()

    wait_edge(0)
    issue_g(0, 0)
    wait_edge(1)
    issue_g(1, 1)

    for k in range(12):
        step(k, k)

    def twelve(i, carry):
        for u in range(12):
            step(12 * i + u, u)
        return carry

    lax.fori_loop(1, (NCH - 5) // 12, twelve, None)

    for k in range(NCH - 5, NCH):
        step(k, k % 12)
    wait_s((NCH - 2) % RS)
    wait_s((NCH - 1) % RS)
    plsc.subcore_barrier()

    for t in range(7):
        pltpu.sync_copy(acc.at[pl.ds(base + t * CK, CK)],
                        out_hbm.at[c, pl.ds(base + t * CK, CK)])
    pltpu.sync_copy(acc.at[pl.ds(base + 7 * CK, RPS - 7 * CK)],
                    out_hbm.at[c, pl.ds(base + 7 * CK, RPS - 7 * CK)])


_spmm_call = functools.partial(
    pl.kernel,
    out_type=jax.ShapeDtypeStruct((NC, P, FEAT), jnp.float32),
    mesh=plsc.VectorSubcoreMesh(core_axis_name="c", subcore_axis_name="s"),
    compiler_params=_SC_PARAMS,
    scratch_types=[
        [pltpu.VMEM((CK, FEAT), jnp.float32) for _ in range(RS)],
        [pltpu.VMEM((CK,), jnp.int32) for _ in range(ES)],
        [pltpu.VMEM((CK,), jnp.int32) for _ in range(ES)],
        [pltpu.VMEM((CK,), jnp.float32) for _ in range(ES)],
        pltpu.VMEM_SHARED((P, FEAT), jnp.float32),
        [pltpu.SemaphoreType.DMA for _ in range(RS)],
        [pltpu.SemaphoreType.DMA for _ in range(RS)],
        [pltpu.SemaphoreType.DMA for _ in range(ES)],
    ],
)(_spmm_body)


def _prologue_tc(xb, w0b, b0b, degb, h_out, hh_out, dinv_out):
    h = jnp.maximum(
        jnp.dot(xb[...], w0b[...], preferred_element_type=jnp.float32)
        + b0b[...], 0.0)
    deg = degb[0, :, 0:1] + degb[1, :, 0:1] + 1.0
    dinv = lax.rsqrt(deg)
    h_out[...] = h
    hh_out[...] = h * dinv
    dinv_out[...] = dinv


def _call_prologue(xp, w0, b0, degp):
    grid = (pl.cdiv(P, BN),)
    return pl.pallas_call(
        _prologue_tc,
        grid=grid,
        in_specs=[
            pl.BlockSpec((BN, FEAT), lambda i: (i, 0)),
            pl.BlockSpec((FEAT, FEAT), lambda i: (0, 0)),
            pl.BlockSpec((1, FEAT), lambda i: (0, 0)),
            pl.BlockSpec((NC, BN, FEAT), lambda i: (0, i, 0)),
        ],
        out_specs=[
            pl.BlockSpec((BN, FEAT), lambda i: (i, 0)),
            pl.BlockSpec((BN, FEAT), lambda i: (i, 0)),
            pl.BlockSpec((BN, 1), lambda i: (i, 0)),
        ],
        out_shape=[
            jax.ShapeDtypeStruct((P, FEAT), jnp.float32),
            jax.ShapeDtypeStruct((P, FEAT), jnp.float32),
            jax.ShapeDtypeStruct((P, 1), jnp.float32),
        ],
    )(xp, w0, b0, degp)


def _layer_tc(spb, hb, h0b, dinvb, wb, hn_out, hhn_out, *, beta):
    s = spb[0] + spb[1]
    dinv = dinvb[...]
    hi = dinv * s + (dinv * dinv) * hb[...]
    support = (1.0 - ALPHA_C) * hi + ALPHA_C * h0b[...]
    out = beta * jnp.dot(support, wb[...],
                         preferred_element_type=jnp.float32) \
        + (1.0 - beta) * support
    hn = jnp.maximum(out, 0.0)
    hn_out[...] = hn
    hhn_out[...] = dinv * hn


def _call_layer(sp, h, h0, dinv, wi, beta):
    grid = (pl.cdiv(P, BN),)
    return pl.pallas_call(
        functools.partial(_layer_tc, beta=beta),
        grid=grid,
        in_specs=[
            pl.BlockSpec((NC, BN, FEAT), lambda i: (0, i, 0)),
            pl.BlockSpec((BN, FEAT), lambda i: (i, 0)),
            pl.BlockSpec((BN, FEAT), lambda i: (i, 0)),
            pl.BlockSpec((BN, 1), lambda i: (i, 0)),
            pl.BlockSpec((FEAT, FEAT), lambda i: (0, 0)),
        ],
        out_specs=[
            pl.BlockSpec((BN, FEAT), lambda i: (i, 0)),
            pl.BlockSpec((BN, FEAT), lambda i: (i, 0)),
        ],
        out_shape=[
            jax.ShapeDtypeStruct((P, FEAT), jnp.float32),
            jax.ShapeDtypeStruct((P, FEAT), jnp.float32),
        ],
    )(sp, h, h0, dinv, wi)


def _final_tc(hb, wob, bob, ob):
    logits = jnp.dot(hb[...], wob[...], preferred_element_type=jnp.float32) \
        + bob[...]
    m = jnp.max(logits, axis=1, keepdims=True)
    lse = jnp.log(jnp.sum(jnp.exp(logits - m), axis=1, keepdims=True)) + m
    ob[...] = logits - lse


def _call_final(h, wout, bout):
    grid = (pl.cdiv(P, BN),)
    return pl.pallas_call(
        _final_tc,
        grid=grid,
        in_specs=[
            pl.BlockSpec((BN, FEAT), lambda i: (i, 0)),
            pl.BlockSpec((FEAT, N_CLASSES), lambda i: (0, 0)),
            pl.BlockSpec((1, N_CLASSES), lambda i: (0, 0)),
        ],
        out_specs=pl.BlockSpec((BN, N_CLASSES), lambda i: (i, 0)),
        out_shape=jax.ShapeDtypeStruct((P, N_CLASSES), jnp.float32),
    )(h, wout, bout)


def kernel(x, edge_index, edge_attr, W0, b0, Ws, Wout, bout):
    xp = jnp.pad(x, ((0, P - N_NODES), (0, 0)))
    row = edge_index[0]
    col = edge_index[1]
    ones = jnp.ones((P, FEAT), jnp.float32)
    degp = _spmm_call(ones, col, col, edge_attr)
    h, hh, dinv = _call_prologue(xp, W0, b0.reshape(1, FEAT), degp)
    h0 = h
    for i in range(N_LAYERS):
        beta = math.log(LAMBDA_C / (i + 1) + 1.0)
        sp = _spmm_call(hh, row, col, edge_attr)
        h, hh = _call_layer(sp, h, h0, dinv, Ws[i], beta)
    out = _call_final(h, Wout, bout.reshape(1, N_CLASSES))
    return out[:N_NODES]


# speedup vs baseline: 16.9012x; 1.1406x over previous
import functools
import math

import jax
import jax.numpy as jnp
from jax import lax
from jax.experimental import pallas as pl
from jax.experimental.pallas import tpu as pltpu
from jax.experimental.pallas import tpu_sc as plsc

N_NODES = 10000
N_EDGES = 320000
FEAT = 128
N_CLASSES = 64
N_LAYERS = 8
ALPHA_C = 0.1
LAMBDA_C = 0.5

NC = 2
NS = 16
NW = NC * NS
P = 10112
RPS = P // NS
EPW = N_EDGES // NW
CK = 80
NCH = EPW // CK
RS = 4
ES = 6

BN = 512

_SC_PARAMS = pltpu.CompilerParams(needs_layout_passes=False)


def _zero_rows(buf, nrows):
    z = jnp.zeros((16,), jnp.float32)
    ncol = buf.shape[1]

    def body(i, carry):
        for j in range(ncol // 16):
            buf[i, pl.ds(j * 16, 16)] = z
        return carry

    lax.fori_loop(0, nrows, body, None)


def _spmm_body(hh_hbm, row_hbm, col_hbm, ew_hbm, out_hbm,
               rows, ri, ci, ewc, acc, sem_g, sem_s, sem_e):
    c = lax.axis_index("c")
    s = lax.axis_index("s")
    w = c * NS + s
    base = s * RPS
    ebase = w * EPW

    def issue_edge(k, e):
        cb = ebase + k * CK
        pltpu.async_copy(row_hbm.at[pl.ds(cb, CK)], ri[e], sem_e[e])
        pltpu.async_copy(col_hbm.at[pl.ds(cb, CK)], ci[e], sem_e[e])
        pltpu.async_copy(ew_hbm.at[pl.ds(cb, CK)], ewc[e], sem_e[e])

    def wait_edge(e):
        pltpu.make_async_copy(row_hbm.at[pl.ds(ebase, CK)], ri[e],
                              sem_e[e]).wait()
        pltpu.make_async_copy(col_hbm.at[pl.ds(ebase, CK)], ci[e],
                              sem_e[e]).wait()
        pltpu.make_async_copy(ew_hbm.at[pl.ds(ebase, CK)], ewc[e],
                              sem_e[e]).wait()

    def issue_g(r, e):
        pltpu.async_copy(hh_hbm.at[ri[e]], rows[r], sem_g[r])

    def wait_g(r):
        pltpu.make_async_copy(hh_hbm.at[ri[0]], rows[r], sem_g[r]).wait()

    def issue_s(r, e):
        pltpu.async_copy(rows[r], acc.at[ci[e]], sem_s[r], add=True)

    def wait_s(r):
        pltpu.make_async_copy(rows[r], acc.at[ci[0]], sem_s[r]).wait()

    def scale(r, e):
        def group(g, cc):
            for u in range(8):
                i = g * 8 + u
                nrm = plsc.load_gather(ewc[e], [jnp.full((16,), i, jnp.int32)])
                for j in range(FEAT // 16):
                    rows[r][i, pl.ds(j * 16, 16)] = \
                        rows[r][i, pl.ds(j * 16, 16)] * nrm
            return cc

        lax.fori_loop(0, CK // 8, group, None)

    def step(k, u):
        m4, m6 = u % RS, u % ES
        wait_g(m4)
        static = isinstance(k, int)
        if not static or k >= 2:
            wait_s((u - 2) % RS)
        if not static or k + 4 <= NCH - 1:
            issue_edge(k + 4, (u + 4) % ES)
        if not static or k + 2 <= NCH - 1:
            wait_edge((u + 2) % ES)
            issue_g((u + 2) % RS, (u + 2) % ES)
        scale(m4, m6)
        issue_s(m4, m6)

    for k in range(4):
        issue_edge(k, k)

    _zero_rows(rows[0], CK)
    for t in range(7):
        pltpu.sync_copy(rows[0], acc.at[pl.ds(base + t * CK, CK)])
    pltpu.sync_copy(rows[0].at[pl.ds(0, RPS - 7 * CK)],
                    acc.at[pl.ds(base + 7 * CK, RPS - 7 * CK)])
    plsc.subcore_barrier()

    wait_edge(0)
    issue_g(0, 0)
    wait_edge(1)
    issue_g(1, 1)

    for k in range(12):
        step(k, k)

    def twelve(i, carry):
        for u in range(12):
            step(12 * i + u, u)
        return carry

    lax.fori_loop(1, (NCH - 5) // 12, twelve, None)

    for k in range(NCH - 5, NCH):
        step(k, k % 12)
    wait_s((NCH - 2) % RS)
    wait_s((NCH - 1) % RS)
    plsc.subcore_barrier()

    for t in range(7):
        pltpu.sync_copy(acc.at[pl.ds(base + t * CK, CK)],
                        out_hbm.at[c, pl.ds(base + t * CK, CK)])
    pltpu.sync_copy(acc.at[pl.ds(base + 7 * CK, RPS - 7 * CK)],
                    out_hbm.at[c, pl.ds(base + 7 * CK, RPS - 7 * CK)])


_spmm_call = functools.partial(
    pl.kernel,
    out_type=jax.ShapeDtypeStruct((NC, P, FEAT), jnp.float32),
    mesh=plsc.VectorSubcoreMesh(core_axis_name="c", subcore_axis_name="s"),
    compiler_params=_SC_PARAMS,
    scratch_types=[
        [pltpu.VMEM((CK, FEAT), jnp.float32) for _ in range(RS)],
        [pltpu.VMEM((CK,), jnp.int32) for _ in range(ES)],
        [pltpu.VMEM((CK,), jnp.int32) for _ in range(ES)],
        [pltpu.VMEM((CK,), jnp.float32) for _ in range(ES)],
        pltpu.VMEM_SHARED((P, FEAT), jnp.float32),
        [pltpu.SemaphoreType.DMA for _ in range(RS)],
        [pltpu.SemaphoreType.DMA for _ in range(RS)],
        [pltpu.SemaphoreType.DMA for _ in range(ES)],
    ],
)(_spmm_body)


def _prologue_tc(xb, w0b, b0b, degb, h_out, hh_out, dinv_out):
    h = jnp.maximum(
        jnp.dot(xb[...], w0b[...], preferred_element_type=jnp.float32)
        + b0b[...], 0.0)
    deg = degb[0, :, 0:1] + degb[1, :, 0:1] + 1.0
    dinv = lax.rsqrt(deg)
    h_out[...] = h
    hh_out[...] = h * dinv
    dinv_out[...] = dinv


def _call_prologue(xp, w0, b0, degp):
    grid = (pl.cdiv(P, BN),)
    return pl.pallas_call(
        _prologue_tc,
        grid=grid,
        in_specs=[
            pl.BlockSpec((BN, FEAT), lambda i: (i, 0)),
            pl.BlockSpec((FEAT, FEAT), lambda i: (0, 0)),
            pl.BlockSpec((1, FEAT), lambda i: (0, 0)),
            pl.BlockSpec((NC, BN, FEAT), lambda i: (0, i, 0)),
        ],
        out_specs=[
            pl.BlockSpec((BN, FEAT), lambda i: (i, 0)),
            pl.BlockSpec((BN, FEAT), lambda i: (i, 0)),
            pl.BlockSpec((BN, 1), lambda i: (i, 0)),
        ],
        out_shape=[
            jax.ShapeDtypeStruct((P, FEAT), jnp.float32),
            jax.ShapeDtypeStruct((P, FEAT), jnp.float32),
            jax.ShapeDtypeStruct((P, 1), jnp.float32),
        ],
    )(xp, w0, b0, degp)


def _layer_tc(spb, hb, h0b, dinvb, wb, hn_out, hhn_out, *, beta):
    s = spb[0] + spb[1]
    dinv = dinvb[...]
    hi = dinv * s + (dinv * dinv) * hb[...]
    support = (1.0 - ALPHA_C) * hi + ALPHA_C * h0b[...]
    out = beta * jnp.dot(support, wb[...],
                         preferred_element_type=jnp.float32) \
        + (1.0 - beta) * support
    hn = jnp.maximum(out, 0.0)
    hn_out[...] = hn
    hhn_out[...] = dinv * hn


def _call_layer(sp, h, h0, dinv, wi, beta):
    grid = (pl.cdiv(P, BN),)
    return pl.pallas_call(
        functools.partial(_layer_tc, beta=beta),
        grid=grid,
        in_specs=[
            pl.BlockSpec((NC, BN, FEAT), lambda i: (0, i, 0)),
            pl.BlockSpec((BN, FEAT), lambda i: (i, 0)),
            pl.BlockSpec((BN, FEAT), lambda i: (i, 0)),
            pl.BlockSpec((BN, 1), lambda i: (i, 0)),
            pl.BlockSpec((FEAT, FEAT), lambda i: (0, 0)),
        ],
        out_specs=[
            pl.BlockSpec((BN, FEAT), lambda i: (i, 0)),
            pl.BlockSpec((BN, FEAT), lambda i: (i, 0)),
        ],
        out_shape=[
            jax.ShapeDtypeStruct((P, FEAT), jnp.float32),
            jax.ShapeDtypeStruct((P, FEAT), jnp.float32),
        ],
    )(sp, h, h0, dinv, wi)


def _final_tc(hb, wob, bob, ob):
    logits = jnp.dot(hb[...], wob[...], preferred_element_type=jnp.float32) \
        + bob[...]
    m = jnp.max(logits, axis=1, keepdims=True)
    lse = jnp.log(jnp.sum(jnp.exp(logits - m), axis=1, keepdims=True)) + m
    ob[...] = logits - lse


def _call_final(h, wout, bout):
    grid = (pl.cdiv(P, BN),)
    return pl.pallas_call(
        _final_tc,
        grid=grid,
        in_specs=[
            pl.BlockSpec((BN, FEAT), lambda i: (i, 0)),
            pl.BlockSpec((FEAT, N_CLASSES), lambda i: (0, 0)),
            pl.BlockSpec((1, N_CLASSES), lambda i: (0, 0)),
        ],
        out_specs=pl.BlockSpec((BN, N_CLASSES), lambda i: (i, 0)),
        out_shape=jax.ShapeDtypeStruct((P, N_CLASSES), jnp.float32),
    )(h, wout, bout)


def kernel(x, edge_index, edge_attr, W0, b0, Ws, Wout, bout):
    xp = jnp.pad(x, ((0, P - N_NODES), (0, 0)))
    row = edge_index[0]
    col = edge_index[1]
    ones = jnp.ones((P, FEAT), jnp.float32)
    degp = _spmm_call(ones, col, col, edge_attr)
    h, hh, dinv = _call_prologue(xp, W0, b0.reshape(1, FEAT), degp)
    h0 = h
    for i in range(N_LAYERS):
        beta = math.log(LAMBDA_C / (i + 1) + 1.0)
        sp = _spmm_call(hh, row, col, edge_attr)
        h, hh = _call_layer(sp, h, h0, dinv, Ws[i], beta)
    out = _call_final(h, Wout, bout.reshape(1, N_CLASSES))
    return out[:N_NODES]


# speedup vs baseline: 17.1183x; 1.0128x over previous
import functools
import math

import jax
import jax.numpy as jnp
from jax import lax
from jax.experimental import pallas as pl
from jax.experimental.pallas import tpu as pltpu
from jax.experimental.pallas import tpu_sc as plsc

N_NODES = 10000
N_EDGES = 320000
FEAT = 128
N_CLASSES = 64
N_LAYERS = 8
ALPHA_C = 0.1
LAMBDA_C = 0.5

NC = 2
NS = 16
NW = NC * NS
P = 10112
RPS = P // NS
EPW = N_EDGES // NW
CK = 80
NCH = EPW // CK
RS = 4
ES = 6

BN = 512

_SC_PARAMS = pltpu.CompilerParams(needs_layout_passes=False)


def _zero_rows(buf, nrows):
    z = jnp.zeros((16,), jnp.float32)
    ncol = buf.shape[1]

    def body(i, carry):
        for j in range(ncol // 16):
            buf[i, pl.ds(j * 16, 16)] = z
        return carry

    lax.fori_loop(0, nrows, body, None)


def _spmm_body(hh_hbm, row_hbm, col_hbm, ew_hbm, out_hbm,
               rows, ri, ci, ewc, acc, sem_g, sem_s, sem_e):
    c = lax.axis_index("c")
    s = lax.axis_index("s")
    w = c * NS + s
    base = s * RPS
    ebase = w * EPW

    def issue_edge(k, e):
        cb = ebase + k * CK
        pltpu.async_copy(row_hbm.at[pl.ds(cb, CK)], ri[e], sem_e[e])
        pltpu.async_copy(col_hbm.at[pl.ds(cb, CK)], ci[e], sem_e[e])
        pltpu.async_copy(ew_hbm.at[pl.ds(cb, CK)], ewc[e], sem_e[e])

    def wait_edge(e):
        pltpu.make_async_copy(row_hbm.at[pl.ds(ebase, CK)], ri[e],
                              sem_e[e]).wait()
        pltpu.make_async_copy(col_hbm.at[pl.ds(ebase, CK)], ci[e],
                              sem_e[e]).wait()
        pltpu.make_async_copy(ew_hbm.at[pl.ds(ebase, CK)], ewc[e],
                              sem_e[e]).wait()

    def issue_g(r, e):
        pltpu.async_copy(hh_hbm.at[ri[e]], rows[r], sem_g[r])

    def wait_g(r):
        pltpu.make_async_copy(hh_hbm.at[ri[0]], rows[r], sem_g[r]).wait()

    def issue_s(r, e):
        pltpu.async_copy(rows[r], acc.at[ci[e]], sem_s[r], add=True)

    def wait_s(r):
        pltpu.make_async_copy(rows[r], acc.at[ci[0]], sem_s[r]).wait()

    def scale(r, e):
        def group(g, cc):
            for u in range(8):
                i = g * 8 + u
                nrm = plsc.load_gather(ewc[e], [jnp.full((16,), i, jnp.int32)])
                for j in range(FEAT // 16):
                    rows[r][i, pl.ds(j * 16, 16)] = \
                        rows[r][i, pl.ds(j * 16, 16)] * nrm
            return cc

        lax.fori_loop(0, CK // 8, group, None)

    def step(k, u):
        m4, m6 = u % RS, u % ES
        wait_g(m4)
        static = isinstance(k, int)
        if not static or k >= 2:
            wait_s((u - 2) % RS)
        if not static or k + 4 <= NCH - 1:
            issue_edge(k + 4, (u + 4) % ES)
        if not static or k + 2 <= NCH - 1:
            wait_edge((u + 2) % ES)
            issue_g((u + 2) % RS, (u + 2) % ES)
        scale(m4, m6)
        issue_s(m4, m6)

    for k in range(4):
        issue_edge(k, k)

    _zero_rows(rows[0], CK)
    for t in range(7):
        pltpu.sync_copy(rows[0], acc.at[pl.ds(base + t * CK, CK)])
    pltpu.sync_copy(rows[0].at[pl.ds(0, RPS - 7 * CK)],
                    acc.at[pl.ds(base + 7 * CK, RPS - 7 * CK)])
    plsc.subcore_barrier()

    wait_edge(0)
    issue_g(0, 0)
    wait_edge(1)
    issue_g(1, 1)

    for k in range(12):
        step(k, k)

    def twelve(i, carry):
        for u in range(12):
            step(12 * i + u, u)
        return carry

    lax.fori_loop(1, (NCH - 5) // 12, twelve, None)

    for k in range(NCH - 5, NCH):
        step(k, k % 12)
    wait_s((NCH - 2) % RS)
    wait_s((NCH - 1) % RS)
    plsc.subcore_barrier()

    for t in range(7):
        pltpu.sync_copy(acc.at[pl.ds(base + t * CK, CK)],
                        out_hbm.at[c, pl.ds(base + t * CK, CK)])
    pltpu.sync_copy(acc.at[pl.ds(base + 7 * CK, RPS - 7 * CK)],
                    out_hbm.at[c, pl.ds(base + 7 * CK, RPS - 7 * CK)])


_spmm_call = functools.partial(
    pl.kernel,
    out_type=jax.ShapeDtypeStruct((NC, P, FEAT), jnp.float32),
    mesh=plsc.VectorSubcoreMesh(core_axis_name="c", subcore_axis_name="s"),
    compiler_params=_SC_PARAMS,
    scratch_types=[
        [pltpu.VMEM((CK, FEAT), jnp.float32) for _ in range(RS)],
        [pltpu.VMEM((CK,), jnp.int32) for _ in range(ES)],
        [pltpu.VMEM((CK,), jnp.int32) for _ in range(ES)],
        [pltpu.VMEM((CK,), jnp.float32) for _ in range(ES)],
        pltpu.VMEM_SHARED((P, FEAT), jnp.float32),
        [pltpu.SemaphoreType.DMA for _ in range(RS)],
        [pltpu.SemaphoreType.DMA for _ in range(RS)],
        [pltpu.SemaphoreType.DMA for _ in range(ES)],
    ],
)(_spmm_body)


def _prologue_tc(xb, w0b, b0b, degb, h_out, hh_out, dinv_out):
    h = jnp.maximum(
        jnp.dot(xb[...], w0b[...], preferred_element_type=jnp.float32)
        + b0b[...], 0.0)
    deg = degb[0, :, 0:1] + degb[1, :, 0:1] + 1.0
    dinv = lax.rsqrt(deg)
    h_out[...] = h
    hh_out[...] = h * dinv
    dinv_out[...] = dinv


def _call_prologue(xp, w0, b0, degp):
    grid = (pl.cdiv(P, BN),)
    return pl.pallas_call(
        _prologue_tc,
        grid=grid,
        in_specs=[
            pl.BlockSpec((BN, FEAT), lambda i: (i, 0)),
            pl.BlockSpec((FEAT, FEAT), lambda i: (0, 0)),
            pl.BlockSpec((1, FEAT), lambda i: (0, 0)),
            pl.BlockSpec((NC, BN, FEAT), lambda i: (0, i, 0)),
        ],
        out_specs=[
            pl.BlockSpec((BN, FEAT), lambda i: (i, 0)),
            pl.BlockSpec((BN, FEAT), lambda i: (i, 0)),
            pl.BlockSpec((BN, 1), lambda i: (i, 0)),
        ],
        out_shape=[
            jax.ShapeDtypeStruct((P, FEAT), jnp.float32),
            jax.ShapeDtypeStruct((P, FEAT), jnp.float32),
            jax.ShapeDtypeStruct((P, 1), jnp.float32),
        ],
    )(xp, w0, b0, degp)


def _layer_tc(spb, hb, h0b, dinvb, wb, hn_out, hhn_out, *, beta):
    s = spb[0] + spb[1]
    dinv = dinvb[...]
    hi = dinv * s + (dinv * dinv) * hb[...]
    support = (1.0 - ALPHA_C) * hi + ALPHA_C * h0b[...]
    out = beta * jnp.dot(support, wb[...],
                         preferred_element_type=jnp.float32) \
        + (1.0 - beta) * support
    hn = jnp.maximum(out, 0.0)
    hn_out[...] = hn
    hhn_out[...] = dinv * hn


def _call_layer(sp, h, h0, dinv, wi, beta):
    grid = (pl.cdiv(P, BN),)
    return pl.pallas_call(
        functools.partial(_layer_tc, beta=beta),
        grid=grid,
        in_specs=[
            pl.BlockSpec((NC, BN, FEAT), lambda i: (0, i, 0)),
            pl.BlockSpec((BN, FEAT), lambda i: (i, 0)),
            pl.BlockSpec((BN, FEAT), lambda i: (i, 0)),
            pl.BlockSpec((BN, 1), lambda i: (i, 0)),
            pl.BlockSpec((FEAT, FEAT), lambda i: (0, 0)),
        ],
        out_specs=[
            pl.BlockSpec((BN, FEAT), lambda i: (i, 0)),
            pl.BlockSpec((BN, FEAT), lambda i: (i, 0)),
        ],
        out_shape=[
            jax.ShapeDtypeStruct((P, FEAT), jnp.float32),
            jax.ShapeDtypeStruct((P, FEAT), jnp.float32),
        ],
    )(sp, h, h0, dinv, wi)


def _last_tc(spb, hb, h0b, dinvb, wb, wob, bob, ob, *, beta):
    s = spb[0] + spb[1]
    dinv = dinvb[...]
    hi = dinv * s + (dinv * dinv) * hb[...]
    support = (1.0 - ALPHA_C) * hi + ALPHA_C * h0b[...]
    out = beta * jnp.dot(support, wb[...],
                         preferred_element_type=jnp.float32) \
        + (1.0 - beta) * support
    hn = jnp.maximum(out, 0.0)
    logits = jnp.dot(hn, wob[...], preferred_element_type=jnp.float32) \
        + bob[...]
    m = jnp.max(logits, axis=1, keepdims=True)
    lse = jnp.log(jnp.sum(jnp.exp(logits - m), axis=1, keepdims=True)) + m
    ob[...] = logits - lse


def _call_last(sp, h, h0, dinv, wi, beta, wout, bout):
    grid = (pl.cdiv(P, BN),)
    return pl.pallas_call(
        functools.partial(_last_tc, beta=beta),
        grid=grid,
        in_specs=[
            pl.BlockSpec((NC, BN, FEAT), lambda i: (0, i, 0)),
            pl.BlockSpec((BN, FEAT), lambda i: (i, 0)),
            pl.BlockSpec((BN, FEAT), lambda i: (i, 0)),
            pl.BlockSpec((BN, 1), lambda i: (i, 0)),
            pl.BlockSpec((FEAT, FEAT), lambda i: (0, 0)),
            pl.BlockSpec((FEAT, N_CLASSES), lambda i: (0, 0)),
            pl.BlockSpec((1, N_CLASSES), lambda i: (0, 0)),
        ],
        out_specs=pl.BlockSpec((BN, N_CLASSES), lambda i: (i, 0)),
        out_shape=jax.ShapeDtypeStruct((P, N_CLASSES), jnp.float32),
    )(sp, h, h0, dinv, wi, wout, bout)


def kernel(x, edge_index, edge_attr, W0, b0, Ws, Wout, bout):
    xp = jnp.pad(x, ((0, P - N_NODES), (0, 0)))
    row = edge_index[0]
    col = edge_index[1]
    ones = jnp.ones((P, FEAT), jnp.float32)
    degp = _spmm_call(ones, col, col, edge_attr)
    h, hh, dinv = _call_prologue(xp, W0, b0.reshape(1, FEAT), degp)
    h0 = h
    for i in range(N_LAYERS - 1):
        beta = math.log(LAMBDA_C / (i + 1) + 1.0)
        sp = _spmm_call(hh, row, col, edge_attr)
        h, hh = _call_layer(sp, h, h0, dinv, Ws[i], beta)
    beta = math.log(LAMBDA_C / N_LAYERS + 1.0)
    sp = _spmm_call(hh, row, col, edge_attr)
    out = _call_last(sp, h, h0, dinv, Ws[N_LAYERS - 1], beta,
                     Wout, bout.reshape(1, N_CLASSES))
    return out[:N_NODES]


# speedup vs baseline: 17.8213x; 1.0411x over previous
import functools
import math

import jax
import jax.numpy as jnp
from jax import lax
from jax.experimental import pallas as pl
from jax.experimental.pallas import tpu as pltpu
from jax.experimental.pallas import tpu_sc as plsc

N_NODES = 10000
N_EDGES = 320000
FEAT = 128
N_CLASSES = 64
N_LAYERS = 8
ALPHA_C = 0.1
LAMBDA_C = 0.5

NC = 2
NS = 16
NW = NC * NS
P = 10112
RPS = P // NS
EPW = N_EDGES // NW
CK = 80
NCH = EPW // CK
RS = 4
ES = 6

BN = 1024

_SC_PARAMS = pltpu.CompilerParams(needs_layout_passes=False)


def _zero_rows(buf, nrows):
    z = jnp.zeros((16,), jnp.float32)
    ncol = buf.shape[1]

    def body(i, carry):
        for j in range(ncol // 16):
            buf[i, pl.ds(j * 16, 16)] = z
        return carry

    lax.fori_loop(0, nrows, body, None)


def _spmm_body(hh_hbm, row_hbm, col_hbm, ew_hbm, out_hbm,
               rows, ri, ci, ewc, acc, sem_g, sem_s, sem_e):
    c = lax.axis_index("c")
    s = lax.axis_index("s")
    w = c * NS + s
    base = s * RPS
    ebase = w * EPW

    def issue_edge(k, e):
        cb = ebase + k * CK
        pltpu.async_copy(row_hbm.at[pl.ds(cb, CK)], ri[e], sem_e[e])
        pltpu.async_copy(col_hbm.at[pl.ds(cb, CK)], ci[e], sem_e[e])
        pltpu.async_copy(ew_hbm.at[pl.ds(cb, CK)], ewc[e], sem_e[e])

    def wait_edge(e):
        pltpu.make_async_copy(row_hbm.at[pl.ds(ebase, CK)], ri[e],
                              sem_e[e]).wait()
        pltpu.make_async_copy(col_hbm.at[pl.ds(ebase, CK)], ci[e],
                              sem_e[e]).wait()
        pltpu.make_async_copy(ew_hbm.at[pl.ds(ebase, CK)], ewc[e],
                              sem_e[e]).wait()

    def issue_g(r, e):
        pltpu.async_copy(hh_hbm.at[ri[e]], rows[r], sem_g[r])

    def wait_g(r):
        pltpu.make_async_copy(hh_hbm.at[ri[0]], rows[r], sem_g[r]).wait()

    def issue_s(r, e):
        pltpu.async_copy(rows[r], acc.at[ci[e]], sem_s[r], add=True)

    def wait_s(r):
        pltpu.make_async_copy(rows[r], acc.at[ci[0]], sem_s[r]).wait()

    def scale(r, e):
        def group(g, cc):
            for u in range(8):
                i = g * 8 + u
                nrm = plsc.load_gather(ewc[e], [jnp.full((16,), i, jnp.int32)])
                for j in range(FEAT // 16):
                    rows[r][i, pl.ds(j * 16, 16)] = \
                        rows[r][i, pl.ds(j * 16, 16)] * nrm
            return cc

        lax.fori_loop(0, CK // 8, group, None)

    def step(k, u):
        m4, m6 = u % RS, u % ES
        wait_g(m4)
        static = isinstance(k, int)
        if not static or k >= 2:
            wait_s((u - 2) % RS)
        if not static or k + 4 <= NCH - 1:
            issue_edge(k + 4, (u + 4) % ES)
        if not static or k + 2 <= NCH - 1:
            wait_edge((u + 2) % ES)
            issue_g((u + 2) % RS, (u + 2) % ES)
        scale(m4, m6)
        issue_s(m4, m6)

    for k in range(4):
        issue_edge(k, k)

    _zero_rows(rows[0], CK)
    for t in range(7):
        pltpu.sync_copy(rows[0], acc.at[pl.ds(base + t * CK, CK)])
    pltpu.sync_copy(rows[0].at[pl.ds(0, RPS - 7 * CK)],
                    acc.at[pl.ds(base + 7 * CK, RPS - 7 * CK)])
    plsc.subcore_barrier()

    wait_edge(0)
    issue_g(0, 0)
    wait_edge(1)
    issue_g(1, 1)

    for k in range(12):
        step(k, k)

    def twelve(i, carry):
        for u in range(12):
            step(12 * i + u, u)
        return carry

    lax.fori_loop(1, (NCH - 5) // 12, twelve, None)

    for k in range(NCH - 5, NCH):
        step(k, k % 12)
    wait_s((NCH - 2) % RS)
    wait_s((NCH - 1) % RS)
    plsc.subcore_barrier()

    for t in range(7):
        pltpu.sync_copy(acc.at[pl.ds(base + t * CK, CK)],
                        out_hbm.at[c, pl.ds(base + t * CK, CK)])
    pltpu.sync_copy(acc.at[pl.ds(base + 7 * CK, RPS - 7 * CK)],
                    out_hbm.at[c, pl.ds(base + 7 * CK, RPS - 7 * CK)])


_spmm_call = functools.partial(
    pl.kernel,
    out_type=jax.ShapeDtypeStruct((NC, P, FEAT), jnp.float32),
    mesh=plsc.VectorSubcoreMesh(core_axis_name="c", subcore_axis_name="s"),
    compiler_params=_SC_PARAMS,
    scratch_types=[
        [pltpu.VMEM((CK, FEAT), jnp.float32) for _ in range(RS)],
        [pltpu.VMEM((CK,), jnp.int32) for _ in range(ES)],
        [pltpu.VMEM((CK,), jnp.int32) for _ in range(ES)],
        [pltpu.VMEM((CK,), jnp.float32) for _ in range(ES)],
        pltpu.VMEM_SHARED((P, FEAT), jnp.float32),
        [pltpu.SemaphoreType.DMA for _ in range(RS)],
        [pltpu.SemaphoreType.DMA for _ in range(RS)],
        [pltpu.SemaphoreType.DMA for _ in range(ES)],
    ],
)(_spmm_body)


def _prologue_tc(xb, w0b, b0b, degb, h_out, hh_out, dinv_out):
    h = jnp.maximum(
        jnp.dot(xb[...], w0b[...], preferred_element_type=jnp.float32)
        + b0b[...], 0.0)
    deg = degb[0, :, 0:1] + degb[1, :, 0:1] + 1.0
    dinv = lax.rsqrt(deg)
    h_out[...] = h
    hh_out[...] = h * dinv
    dinv_out[...] = dinv


def _call_prologue(xp, w0, b0, degp):
    grid = (pl.cdiv(P, BN),)
    return pl.pallas_call(
        _prologue_tc,
        grid=grid,
        in_specs=[
            pl.BlockSpec((BN, FEAT), lambda i: (i, 0)),
            pl.BlockSpec((FEAT, FEAT), lambda i: (0, 0)),
            pl.BlockSpec((1, FEAT), lambda i: (0, 0)),
            pl.BlockSpec((NC, BN, FEAT), lambda i: (0, i, 0)),
        ],
        out_specs=[
            pl.BlockSpec((BN, FEAT), lambda i: (i, 0)),
            pl.BlockSpec((BN, FEAT), lambda i: (i, 0)),
            pl.BlockSpec((BN, 1), lambda i: (i, 0)),
        ],
        out_shape=[
            jax.ShapeDtypeStruct((P, FEAT), jnp.float32),
            jax.ShapeDtypeStruct((P, FEAT), jnp.float32),
            jax.ShapeDtypeStruct((P, 1), jnp.float32),
        ],
    )(xp, w0, b0, degp)


def _layer_tc(spb, hb, h0b, dinvb, wb, hn_out, hhn_out, *, beta):
    s = spb[0] + spb[1]
    dinv = dinvb[...]
    hi = dinv * s + (dinv * dinv) * hb[...]
    support = (1.0 - ALPHA_C) * hi + ALPHA_C * h0b[...]
    out = beta * jnp.dot(support, wb[...],
                         preferred_element_type=jnp.float32) \
        + (1.0 - beta) * support
    hn = jnp.maximum(out, 0.0)
    hn_out[...] = hn
    hhn_out[...] = dinv * hn


def _call_layer(sp, h, h0, dinv, wi, beta):
    grid = (pl.cdiv(P, BN),)
    return pl.pallas_call(
        functools.partial(_layer_tc, beta=beta),
        grid=grid,
        in_specs=[
            pl.BlockSpec((NC, BN, FEAT), lambda i: (0, i, 0)),
            pl.BlockSpec((BN, FEAT), lambda i: (i, 0)),
            pl.BlockSpec((BN, FEAT), lambda i: (i, 0)),
            pl.BlockSpec((BN, 1), lambda i: (i, 0)),
            pl.BlockSpec((FEAT, FEAT), lambda i: (0, 0)),
        ],
        out_specs=[
            pl.BlockSpec((BN, FEAT), lambda i: (i, 0)),
            pl.BlockSpec((BN, FEAT), lambda i: (i, 0)),
        ],
        out_shape=[
            jax.ShapeDtypeStruct((P, FEAT), jnp.float32),
            jax.ShapeDtypeStruct((P, FEAT), jnp.float32),
        ],
    )(sp, h, h0, dinv, wi)


def _last_tc(spb, hb, h0b, dinvb, wb, wob, bob, ob, *, beta):
    s = spb[0] + spb[1]
    dinv = dinvb[...]
    hi = dinv * s + (dinv * dinv) * hb[...]
    support = (1.0 - ALPHA_C) * hi + ALPHA_C * h0b[...]
    out = beta * jnp.dot(support, wb[...],
                         preferred_element_type=jnp.float32) \
        + (1.0 - beta) * support
    hn = jnp.maximum(out, 0.0)
    logits = jnp.dot(hn, wob[...], preferred_element_type=jnp.float32) \
        + bob[...]
    m = jnp.max(logits, axis=1, keepdims=True)
    lse = jnp.log(jnp.sum(jnp.exp(logits - m), axis=1, keepdims=True)) + m
    ob[...] = logits - lse


def _call_last(sp, h, h0, dinv, wi, beta, wout, bout):
    grid = (pl.cdiv(P, BN),)
    return pl.pallas_call(
        functools.partial(_last_tc, beta=beta),
        grid=grid,
        in_specs=[
            pl.BlockSpec((NC, BN, FEAT), lambda i: (0, i, 0)),
            pl.BlockSpec((BN, FEAT), lambda i: (i, 0)),
            pl.BlockSpec((BN, FEAT), lambda i: (i, 0)),
            pl.BlockSpec((BN, 1), lambda i: (i, 0)),
            pl.BlockSpec((FEAT, FEAT), lambda i: (0, 0)),
            pl.BlockSpec((FEAT, N_CLASSES), lambda i: (0, 0)),
            pl.BlockSpec((1, N_CLASSES), lambda i: (0, 0)),
        ],
        out_specs=pl.BlockSpec((BN, N_CLASSES), lambda i: (i, 0)),
        out_shape=jax.ShapeDtypeStruct((P, N_CLASSES), jnp.float32),
    )(sp, h, h0, dinv, wi, wout, bout)


def kernel(x, edge_index, edge_attr, W0, b0, Ws, Wout, bout):
    xp = jnp.pad(x, ((0, P - N_NODES), (0, 0)))
    row = edge_index[0]
    col = edge_index[1]
    ones = jnp.ones((P, FEAT), jnp.float32)
    degp = _spmm_call(ones, col, col, edge_attr)
    h, hh, dinv = _call_prologue(xp, W0, b0.reshape(1, FEAT), degp)
    h0 = h
    for i in range(N_LAYERS - 1):
        beta = math.log(LAMBDA_C / (i + 1) + 1.0)
        sp = _spmm_call(hh, row, col, edge_attr)
        h, hh = _call_layer(sp, h, h0, dinv, Ws[i], beta)
    beta = math.log(LAMBDA_C / N_LAYERS + 1.0)
    sp = _spmm_call(hh, row, col, edge_attr)
    out = _call_last(sp, h, h0, dinv, Ws[N_LAYERS - 1], beta,
                     Wout, bout.reshape(1, N_CLASSES))
    return out[:N_NODES]


# speedup vs baseline: 18.2742x; 1.0254x over previous
import functools
import math

import jax
import jax.numpy as jnp
from jax import lax
from jax.experimental import pallas as pl
from jax.experimental.pallas import tpu as pltpu
from jax.experimental.pallas import tpu_sc as plsc

N_NODES = 10000
N_EDGES = 320000
FEAT = 128
N_CLASSES = 64
N_LAYERS = 8
ALPHA_C = 0.1
LAMBDA_C = 0.5

NC = 2
NS = 16
NW = NC * NS
P = 10112
RPS = P // NS
EPW = N_EDGES // NW
CK = 80
NCH = EPW // CK
RS = 4
ES = 6

BN = 1024

_SC_PARAMS = pltpu.CompilerParams(needs_layout_passes=False)


def _zero_rows(buf, nrows):
    z = jnp.zeros((16,), jnp.float32)
    ncol = buf.shape[1]

    def body(i, carry):
        for j in range(ncol // 16):
            buf[i, pl.ds(j * 16, 16)] = z
        return carry

    lax.fori_loop(0, nrows, body, None)


def _spmm_body(hh_hbm, row_hbm, col_hbm, ew_hbm, out_hbm,
               rows, ri, ci, ewc, acc, sem_g, sem_s, sem_e, *, fill=False):
    c = lax.axis_index("c")
    s = lax.axis_index("s")
    w = c * NS + s
    base = s * RPS
    ebase = w * EPW

    def issue_edge(k, e):
        cb = ebase + k * CK
        pltpu.async_copy(row_hbm.at[pl.ds(cb, CK)], ri[e], sem_e[e])
        pltpu.async_copy(col_hbm.at[pl.ds(cb, CK)], ci[e], sem_e[e])
        pltpu.async_copy(ew_hbm.at[pl.ds(cb, CK)], ewc[e], sem_e[e])

    def wait_edge(e):
        pltpu.make_async_copy(row_hbm.at[pl.ds(ebase, CK)], ri[e],
                              sem_e[e]).wait()
        pltpu.make_async_copy(col_hbm.at[pl.ds(ebase, CK)], ci[e],
                              sem_e[e]).wait()
        pltpu.make_async_copy(ew_hbm.at[pl.ds(ebase, CK)], ewc[e],
                              sem_e[e]).wait()

    def issue_g(r, e):
        pltpu.async_copy(hh_hbm.at[ri[e]], rows[r], sem_g[r])

    def wait_g(r):
        pltpu.make_async_copy(hh_hbm.at[ri[0]], rows[r], sem_g[r]).wait()

    def issue_s(r, e):
        pltpu.async_copy(rows[r], acc.at[ci[e]], sem_s[r], add=True)

    def wait_s(r):
        pltpu.make_async_copy(rows[r], acc.at[ci[0]], sem_s[r]).wait()

    def scale(r, e):
        def group(g, cc):
            for u in range(8):
                i = g * 8 + u
                nrm = plsc.load_gather(ewc[e], [jnp.full((16,), i, jnp.int32)])
                for j in range(FEAT // 16):
                    if fill:
                        rows[r][i, pl.ds(j * 16, 16)] = nrm
                    else:
                        rows[r][i, pl.ds(j * 16, 16)] = \
                            rows[r][i, pl.ds(j * 16, 16)] * nrm
            return cc

        lax.fori_loop(0, CK // 8, group, None)

    def step(k, u):
        m4, m6 = u % RS, u % ES
        if not fill:
            wait_g(m4)
        static = isinstance(k, int)
        if not static or k >= 2:
            wait_s((u - 2) % RS)
        if not static or k + 4 <= NCH - 1:
            issue_edge(k + 4, (u + 4) % ES)
        if not static or k + 2 <= NCH - 1:
            wait_edge((u + 2) % ES)
            if not fill:
                issue_g((u + 2) % RS, (u + 2) % ES)
        scale(m4, m6)
        issue_s(m4, m6)

    for k in range(4):
        issue_edge(k, k)

    _zero_rows(rows[0], CK)
    for t in range(7):
        pltpu.sync_copy(rows[0], acc.at[pl.ds(base + t * CK, CK)])
    pltpu.sync_copy(rows[0].at[pl.ds(0, RPS - 7 * CK)],
                    acc.at[pl.ds(base + 7 * CK, RPS - 7 * CK)])
    plsc.subcore_barrier()

    wait_edge(0)
    if not fill:
        issue_g(0, 0)
    wait_edge(1)
    if not fill:
        issue_g(1, 1)

    for k in range(12):
        step(k, k)

    def twelve(i, carry):
        for u in range(12):
            step(12 * i + u, u)
        return carry

    lax.fori_loop(1, (NCH - 5) // 12, twelve, None)

    for k in range(NCH - 5, NCH):
        step(k, k % 12)
    wait_s((NCH - 2) % RS)
    wait_s((NCH - 1) % RS)
    plsc.subcore_barrier()

    for t in range(7):
        pltpu.sync_copy(acc.at[pl.ds(base + t * CK, CK)],
                        out_hbm.at[c, pl.ds(base + t * CK, CK)])
    pltpu.sync_copy(acc.at[pl.ds(base + 7 * CK, RPS - 7 * CK)],
                    out_hbm.at[c, pl.ds(base + 7 * CK, RPS - 7 * CK)])


_SPMM_SCRATCH = [
    [pltpu.VMEM((CK, FEAT), jnp.float32) for _ in range(RS)],
    [pltpu.VMEM((CK,), jnp.int32) for _ in range(ES)],
    [pltpu.VMEM((CK,), jnp.int32) for _ in range(ES)],
    [pltpu.VMEM((CK,), jnp.float32) for _ in range(ES)],
    pltpu.VMEM_SHARED((P, FEAT), jnp.float32),
    [pltpu.SemaphoreType.DMA for _ in range(RS)],
    [pltpu.SemaphoreType.DMA for _ in range(RS)],
    [pltpu.SemaphoreType.DMA for _ in range(ES)],
]

_spmm_call = functools.partial(
    pl.kernel,
    out_type=jax.ShapeDtypeStruct((NC, P, FEAT), jnp.float32),
    mesh=plsc.VectorSubcoreMesh(core_axis_name="c", subcore_axis_name="s"),
    compiler_params=_SC_PARAMS,
    scratch_types=_SPMM_SCRATCH,
)(_spmm_body)

_deg_call = functools.partial(
    pl.kernel,
    out_type=jax.ShapeDtypeStruct((NC, P, FEAT), jnp.float32),
    mesh=plsc.VectorSubcoreMesh(core_axis_name="c", subcore_axis_name="s"),
    compiler_params=_SC_PARAMS,
    scratch_types=_SPMM_SCRATCH,
)(functools.partial(_spmm_body, fill=True))


def _prologue_tc(xb, w0b, b0b, degb, h_out, hh_out, dinv_out):
    h = jnp.maximum(
        jnp.dot(xb[...], w0b[...], preferred_element_type=jnp.float32)
        + b0b[...], 0.0)
    deg = degb[0, :, 0:1] + degb[1, :, 0:1] + 1.0
    dinv = lax.rsqrt(deg)
    h_out[...] = h
    hh_out[...] = h * dinv
    dinv_out[...] = dinv


def _call_prologue(xp, w0, b0, degp):
    grid = (pl.cdiv(P, BN),)
    return pl.pallas_call(
        _prologue_tc,
        grid=grid,
        in_specs=[
            pl.BlockSpec((BN, FEAT), lambda i: (i, 0)),
            pl.BlockSpec((FEAT, FEAT), lambda i: (0, 0)),
            pl.BlockSpec((1, FEAT), lambda i: (0, 0)),
            pl.BlockSpec((NC, BN, FEAT), lambda i: (0, i, 0)),
        ],
        out_specs=[
            pl.BlockSpec((BN, FEAT), lambda i: (i, 0)),
            pl.BlockSpec((BN, FEAT), lambda i: (i, 0)),
            pl.BlockSpec((BN, 1), lambda i: (i, 0)),
        ],
        out_shape=[
            jax.ShapeDtypeStruct((P, FEAT), jnp.float32),
            jax.ShapeDtypeStruct((P, FEAT), jnp.float32),
            jax.ShapeDtypeStruct((P, 1), jnp.float32),
        ],
    )(xp, w0, b0, degp)


def _layer_tc(spb, hb, h0b, dinvb, wb, hn_out, hhn_out, *, beta):
    s = spb[0] + spb[1]
    dinv = dinvb[...]
    hi = dinv * s + (dinv * dinv) * hb[...]
    support = (1.0 - ALPHA_C) * hi + ALPHA_C * h0b[...]
    out = beta * jnp.dot(support, wb[...],
                         preferred_element_type=jnp.float32) \
        + (1.0 - beta) * support
    hn = jnp.maximum(out, 0.0)
    hn_out[...] = hn
    hhn_out[...] = dinv * hn


def _call_layer(sp, h, h0, dinv, wi, beta):
    grid = (pl.cdiv(P, BN),)
    return pl.pallas_call(
        functools.partial(_layer_tc, beta=beta),
        grid=grid,
        in_specs=[
            pl.BlockSpec((NC, BN, FEAT), lambda i: (0, i, 0)),
            pl.BlockSpec((BN, FEAT), lambda i: (i, 0)),
            pl.BlockSpec((BN, FEAT), lambda i: (i, 0)),
            pl.BlockSpec((BN, 1), lambda i: (i, 0)),
            pl.BlockSpec((FEAT, FEAT), lambda i: (0, 0)),
        ],
        out_specs=[
            pl.BlockSpec((BN, FEAT), lambda i: (i, 0)),
            pl.BlockSpec((BN, FEAT), lambda i: (i, 0)),
        ],
        out_shape=[
            jax.ShapeDtypeStruct((P, FEAT), jnp.float32),
            jax.ShapeDtypeStruct((P, FEAT), jnp.float32),
        ],
    )(sp, h, h0, dinv, wi)


def _last_tc(spb, hb, h0b, dinvb, wb, wob, bob, ob, *, beta):
    s = spb[0] + spb[1]
    dinv = dinvb[...]
    hi = dinv * s + (dinv * dinv) * hb[...]
    support = (1.0 - ALPHA_C) * hi + ALPHA_C * h0b[...]
    out = beta * jnp.dot(support, wb[...],
                         preferred_element_type=jnp.float32) \
        + (1.0 - beta) * support
    hn = jnp.maximum(out, 0.0)
    logits = jnp.dot(hn, wob[...], preferred_element_type=jnp.float32) \
        + bob[...]
    m = jnp.max(logits, axis=1, keepdims=True)
    lse = jnp.log(jnp.sum(jnp.exp(logits - m), axis=1, keepdims=True)) + m
    ob[...] = logits - lse


def _call_last(sp, h, h0, dinv, wi, beta, wout, bout):
    grid = (pl.cdiv(P, BN),)
    return pl.pallas_call(
        functools.partial(_last_tc, beta=beta),
        grid=grid,
        in_specs=[
            pl.BlockSpec((NC, BN, FEAT), lambda i: (0, i, 0)),
            pl.BlockSpec((BN, FEAT), lambda i: (i, 0)),
            pl.BlockSpec((BN, FEAT), lambda i: (i, 0)),
            pl.BlockSpec((BN, 1), lambda i: (i, 0)),
            pl.BlockSpec((FEAT, FEAT), lambda i: (0, 0)),
            pl.BlockSpec((FEAT, N_CLASSES), lambda i: (0, 0)),
            pl.BlockSpec((1, N_CLASSES), lambda i: (0, 0)),
        ],
        out_specs=pl.BlockSpec((BN, N_CLASSES), lambda i: (i, 0)),
        out_shape=jax.ShapeDtypeStruct((P, N_CLASSES), jnp.float32),
    )(sp, h, h0, dinv, wi, wout, bout)


def kernel(x, edge_index, edge_attr, W0, b0, Ws, Wout, bout):
    xp = jnp.pad(x, ((0, P - N_NODES), (0, 0)))
    row = edge_index[0]
    col = edge_index[1]
    degp = _deg_call(xp, col, col, edge_attr)
    h, hh, dinv = _call_prologue(xp, W0, b0.reshape(1, FEAT), degp)
    h0 = h
    for i in range(N_LAYERS - 1):
        beta = math.log(LAMBDA_C / (i + 1) + 1.0)
        sp = _spmm_call(hh, row, col, edge_attr)
        h, hh = _call_layer(sp, h, h0, dinv, Ws[i], beta)
    beta = math.log(LAMBDA_C / N_LAYERS + 1.0)
    sp = _spmm_call(hh, row, col, edge_attr)
    out = _call_last(sp, h, h0, dinv, Ws[N_LAYERS - 1], beta,
                     Wout, bout.reshape(1, N_CLASSES))
    return out[:N_NODES]
